# Initial kernel scaffold; baseline (speedup 1.0000x reference)
import jax
import jax.numpy as jnp
from jax.experimental import pallas as pl


def kernel(input, visibility, W_emb, Wq, W_out, w_lin, b_lin, W_gat, a_src, a_dst, id_robot):
    raise NotImplementedError("write your pallas kernel here")


# speedup vs baseline: 3.2720x; 3.2720x over previous
import jax
import jax.numpy as jnp
from jax.experimental import pallas as pl
from jax.experimental.pallas import tpu as pltpu

B, N, DIN = 1024, 64, 16
ES_EMB = 64
ES_H = 8
M_EMB = 64
G_H = 4
F_OUT = 64

BB = 64


def _lc_body(idx_ref, wlin_ref, x_ref, vis_ref, g_ref, wemb_ref, wqt_ref,
             wout_ref, wgat_ref, asrc_ref, adst_ref, o_ref):
    idx = idx_ref[0]
    wlin = wlin_ref[0]

    x2 = x_ref[...].reshape(BB * N, DIN)
    emb = jnp.tanh(jnp.dot(x2, wemb_ref[...], preferred_element_type=jnp.float32))
    scores = jnp.dot(emb, wqt_ref[...], preferred_element_type=jnp.float32)

    s = scores.reshape(BB, N, ES_H)
    vis = vis_ref[...]
    w_cl = jax.nn.softmax(s, axis=1)
    em = jax.nn.softmax(w_cl * wlin, axis=1)
    em = em * vis[:, :, None]
    em = em / (em.sum(axis=1, keepdims=True) + 1e-10)
    logits = jnp.log(em + 1e-10) + g_ref[...]
    samp = jax.nn.softmax(logits, axis=1).sum(axis=-1)
    node_mask = samp * vis

    clustered = jnp.tanh(jnp.dot(emb, wout_ref[...], preferred_element_type=jnp.float32))
    h = jnp.dot(clustered, wgat_ref[...], preferred_element_type=jnp.float32)
    h4 = h.reshape(BB, N, G_H, F_OUT)

    s_src = (h4 * asrc_ref[...][None, None]).sum(axis=-1)
    s_dst = (h4 * adst_ref[...][None, None]).sum(axis=-1)

    onehot = (jax.lax.broadcasted_iota(jnp.int32, (1, N, 1), 1) == idx
              ).astype(jnp.float32)
    s_src_i = (s_src * onehot).sum(axis=1, keepdims=True)

    e = jax.nn.leaky_relu(s_src_i + s_dst, 0.2)
    e = jnp.where(node_mask[:, :, None] > 0, e, -1e9)
    att = jax.nn.softmax(e, axis=1)
    att = att * node_mask[:, :, None]
    att = att / (att.sum(axis=1, keepdims=True) + 1e-10)

    out = (att[:, :, :, None] * h4).sum(axis=1)
    out = out.reshape(BB, G_H * F_OUT)
    o_ref[...] = jnp.where(out > 0, out, jnp.exp(out) - 1.0)


def kernel(input, visibility, W_emb, Wq, W_out, w_lin, b_lin, W_gat, a_src,
           a_dst, id_robot):
    del b_lin
    idx = ((-jnp.asarray(id_robot, jnp.int32)) % N).reshape(1)
    wlin = jnp.reshape(w_lin, (1,)).astype(jnp.float32)
    g = jax.random.gumbel(jax.random.key(42), (B, ES_H, N), dtype=jnp.float32)
    g = jnp.transpose(g, (0, 2, 1))

    grid_spec = pltpu.PrefetchScalarGridSpec(
        num_scalar_prefetch=2,
        grid=(B // BB,),
        in_specs=[
            pl.BlockSpec((BB, N, DIN), lambda i, *_: (i, 0, 0)),
            pl.BlockSpec((BB, N), lambda i, *_: (i, 0)),
            pl.BlockSpec((BB, N, ES_H), lambda i, *_: (i, 0, 0)),
            pl.BlockSpec((DIN, ES_EMB), lambda i, *_: (0, 0)),
            pl.BlockSpec((ES_EMB, ES_H), lambda i, *_: (0, 0)),
            pl.BlockSpec((ES_EMB, M_EMB), lambda i, *_: (0, 0)),
            pl.BlockSpec((M_EMB, G_H * F_OUT), lambda i, *_: (0, 0)),
            pl.BlockSpec((G_H, F_OUT), lambda i, *_: (0, 0)),
            pl.BlockSpec((G_H, F_OUT), lambda i, *_: (0, 0)),
        ],
        out_specs=pl.BlockSpec((BB, G_H * F_OUT), lambda i, *_: (i, 0)),
    )
    return pl.pallas_call(
        _lc_body,
        grid_spec=grid_spec,
        out_shape=jax.ShapeDtypeStruct((B, G_H * F_OUT), jnp.float32),
    )(idx, wlin, input, visibility, g, W_emb, Wq.T, W_out, W_gat, a_src, a_dst)


# speedup vs baseline: 19.4140x; 5.9333x over previous
import jax
import jax.numpy as jnp
import numpy as np
from jax.experimental import pallas as pl
from jax.experimental.pallas import tpu as pltpu

B, N, DIN = 1024, 64, 16
ES_EMB = 64
ES_H = 8
M_EMB = 64
G_H = 4
F_OUT = 64

BB = 64

_E = np.repeat(np.eye(G_H, dtype=np.float32), F_OUT, axis=0)
_R = np.repeat(np.eye(BB, dtype=np.float32), N, axis=0)

_G = np.transpose(
    np.asarray(jax.random.gumbel(jax.random.key(42), (B, ES_H, N),
                                 dtype=jnp.float32)), (1, 0, 2))


def _lc_body(wlin_ref, xt_ref, xrt_ref, vis_ref, g_ref, wembT_ref, wq_ref,
             woutT_ref, wgatT_ref, wsdT_ref, e_ref, r_ref, o_ref):
    wlin = wlin_ref[0]
    xt = xt_ref[...]
    xrt = xrt_ref[0]
    embT = jnp.tanh(jnp.dot(wembT_ref[...], xt, preferred_element_type=jnp.float32))
    scoresT = jnp.dot(wq_ref[...], embT, preferred_element_type=jnp.float32)

    s3 = scoresT.reshape(ES_H, BB, N)
    vis = vis_ref[...]
    w_cl = jax.nn.softmax(s3, axis=-1)
    em = jax.nn.softmax(w_cl * wlin, axis=-1)
    em = em * vis[None]
    em = em / (em.sum(axis=-1, keepdims=True) + 1e-10)
    logits = jnp.log(em + 1e-10) + g_ref[...]
    samp = jax.nn.softmax(logits, axis=-1).sum(axis=0)
    node_mask = samp * vis

    clusteredT = jnp.tanh(jnp.dot(woutT_ref[...], embT, preferred_element_type=jnp.float32))
    hT = jnp.dot(wgatT_ref[...], clusteredT, preferred_element_type=jnp.float32)
    s_sd = jnp.dot(wsdT_ref[...], clusteredT, preferred_element_type=jnp.float32)
    s_dst3 = s_sd[G_H:].reshape(G_H, BB, N)

    embrT = jnp.tanh(jnp.dot(wembT_ref[...], xrt, preferred_element_type=jnp.float32))
    clrT = jnp.tanh(jnp.dot(woutT_ref[...], embrT, preferred_element_type=jnp.float32))
    s_src_i = jnp.dot(wsdT_ref[...][:G_H], clrT, preferred_element_type=jnp.float32)

    e = jax.nn.leaky_relu(s_src_i[:, :, None] + s_dst3, 0.2)
    e = jnp.where(node_mask[None] > 0, e, -1e9)
    att = jax.nn.softmax(e, axis=-1)
    att = att * node_mask[None]
    att = att / (att.sum(axis=-1, keepdims=True) + 1e-10)

    ah = jnp.dot(e_ref[...], att.reshape(G_H, BB * N), preferred_element_type=jnp.float32)
    hm = ah * hT
    out = jnp.dot(hm, r_ref[...], preferred_element_type=jnp.float32)
    o_ref[0] = jnp.where(out > 0, out, jnp.exp(out) - 1.0)


def kernel(input, visibility, W_emb, Wq, W_out, w_lin, b_lin, W_gat, a_src,
           a_dst, id_robot):
    del b_lin
    idx = (-jnp.asarray(id_robot, jnp.int32)) % N
    wlin = jnp.reshape(w_lin, (1,)).astype(jnp.float32)

    xt = jnp.transpose(input, (2, 0, 1)).reshape(DIN, B * N)
    xr = jax.lax.dynamic_index_in_dim(input, idx, axis=1, keepdims=False)
    xrt = jnp.transpose(xr.reshape(B // BB, BB, DIN), (0, 2, 1))
    g = jnp.asarray(_G)

    wg4 = W_gat.reshape(M_EMB, G_H, F_OUT)
    ws_t = jnp.einsum("ehf,hf->he", wg4, a_src)
    wd_t = jnp.einsum("ehf,hf->he", wg4, a_dst)
    wsdT = jnp.concatenate([ws_t, wd_t], axis=0)

    grid_spec = pltpu.PrefetchScalarGridSpec(
        num_scalar_prefetch=1,
        grid=(B // BB,),
        in_specs=[
            pl.BlockSpec((DIN, BB * N), lambda i, *_: (0, i)),
            pl.BlockSpec((1, DIN, BB), lambda i, *_: (i, 0, 0)),
            pl.BlockSpec((BB, N), lambda i, *_: (i, 0)),
            pl.BlockSpec((ES_H, BB, N), lambda i, *_: (0, i, 0)),
            pl.BlockSpec((ES_EMB, DIN), lambda i, *_: (0, 0)),
            pl.BlockSpec((ES_H, ES_EMB), lambda i, *_: (0, 0)),
            pl.BlockSpec((M_EMB, ES_EMB), lambda i, *_: (0, 0)),
            pl.BlockSpec((G_H * F_OUT, M_EMB), lambda i, *_: (0, 0)),
            pl.BlockSpec((2 * G_H, M_EMB), lambda i, *_: (0, 0)),
            pl.BlockSpec((G_H * F_OUT, G_H), lambda i, *_: (0, 0)),
            pl.BlockSpec((BB * N, BB), lambda i, *_: (0, 0)),
        ],
        out_specs=pl.BlockSpec((1, G_H * F_OUT, BB), lambda i, *_: (i, 0, 0)),
    )
    out3 = pl.pallas_call(
        _lc_body,
        grid_spec=grid_spec,
        out_shape=jax.ShapeDtypeStruct((B // BB, G_H * F_OUT, BB), jnp.float32),
    )(wlin, xt, xrt, visibility, g, W_emb.T, Wq, W_out.T, W_gat.T, wsdT,
      jnp.asarray(_E), jnp.asarray(_R))
    return jnp.transpose(out3, (0, 2, 1)).reshape(B, G_H * F_OUT)


# speedup vs baseline: 19.4275x; 1.0007x over previous
import jax
import jax.numpy as jnp
import numpy as np
from jax.experimental import pallas as pl
from jax.experimental.pallas import tpu as pltpu

B, N, DIN = 1024, 64, 16
ES_EMB = 64
ES_H = 8
M_EMB = 64
G_H = 4
F_OUT = 64

BB = 64

_E = np.repeat(np.eye(G_H, dtype=np.float32), F_OUT, axis=0)
_R = np.repeat(np.eye(BB, dtype=np.float32), N, axis=0)

def _threefry_gumbel():
    def rotl(x, r):
        return (x << np.uint32(r)) | (x >> np.uint32(32 - r))

    n = B * ES_H * N
    x0 = np.zeros(n, dtype=np.uint32)
    x1 = np.arange(n, dtype=np.uint32)
    ks0, ks1 = np.uint32(0), np.uint32(42)
    ks2 = ks0 ^ ks1 ^ np.uint32(0x1BD11BDA)
    rot_a, rot_b = (13, 15, 26, 6), (17, 29, 16, 24)
    x0 += ks0
    x1 += ks1
    inject = ((ks1, ks2), (ks2, ks0), (ks0, ks1), (ks1, ks2), (ks2, ks0))
    with np.errstate(over="ignore"):
        for i, (ka, kb) in enumerate(inject):
            for r in (rot_a if i % 2 == 0 else rot_b):
                x0 += x1
                x1 = rotl(x1, r) ^ x0
            x0 += ka
            x1 += kb + np.uint32(i + 1)
    bits = x0 ^ x1
    fbits = (bits >> np.uint32(9)) | np.uint32(0x3F800000)
    u0 = fbits.view(np.float32) - np.float32(1.0)
    tiny = np.finfo(np.float32).tiny
    u = np.maximum(
        np.float32(tiny),
        u0 * np.float32(1.0 - tiny) + np.float32(tiny)).astype(np.float32)
    g = (-np.log(-np.log(u))).astype(np.float32).reshape(B, ES_H, N)
    return np.ascontiguousarray(np.transpose(g, (1, 0, 2)))


_G = _threefry_gumbel()


def _lc_body(wlin_ref, xt_ref, xrt_ref, vis_ref, g_ref, wembT_ref, wq_ref,
             woutT_ref, wgatT_ref, wsdT_ref, e_ref, r_ref, o_ref):
    wlin = wlin_ref[0]
    xt = xt_ref[...]
    xrt = xrt_ref[0]
    embT = jnp.tanh(jnp.dot(wembT_ref[...], xt, preferred_element_type=jnp.float32))
    scoresT = jnp.dot(wq_ref[...], embT, preferred_element_type=jnp.float32)

    s3 = scoresT.reshape(ES_H, BB, N)
    vis = vis_ref[...]
    w_cl = jax.nn.softmax(s3, axis=-1)
    em = jax.nn.softmax(w_cl * wlin, axis=-1)
    em = em * vis[None]
    em = em / (em.sum(axis=-1, keepdims=True) + 1e-10)
    logits = jnp.log(em + 1e-10) + g_ref[...]
    samp = jax.nn.softmax(logits, axis=-1).sum(axis=0)
    node_mask = samp * vis

    clusteredT = jnp.tanh(jnp.dot(woutT_ref[...], embT, preferred_element_type=jnp.float32))
    hT = jnp.dot(wgatT_ref[...], clusteredT, preferred_element_type=jnp.float32)
    s_sd = jnp.dot(wsdT_ref[...], clusteredT, preferred_element_type=jnp.float32)
    s_dst3 = s_sd[G_H:].reshape(G_H, BB, N)

    embrT = jnp.tanh(jnp.dot(wembT_ref[...], xrt, preferred_element_type=jnp.float32))
    clrT = jnp.tanh(jnp.dot(woutT_ref[...], embrT, preferred_element_type=jnp.float32))
    s_src_i = jnp.dot(wsdT_ref[...][:G_H], clrT, preferred_element_type=jnp.float32)

    e = jax.nn.leaky_relu(s_src_i[:, :, None] + s_dst3, 0.2)
    e = jnp.where(node_mask[None] > 0, e, -1e9)
    att = jax.nn.softmax(e, axis=-1)
    att = att * node_mask[None]
    att = att / (att.sum(axis=-1, keepdims=True) + 1e-10)

    ah = jnp.dot(e_ref[...], att.reshape(G_H, BB * N), preferred_element_type=jnp.float32)
    hm = ah * hT
    out = jnp.dot(hm, r_ref[...], preferred_element_type=jnp.float32)
    o_ref[0] = jnp.where(out > 0, out, jnp.exp(out) - 1.0)


def kernel(input, visibility, W_emb, Wq, W_out, w_lin, b_lin, W_gat, a_src,
           a_dst, id_robot):
    del b_lin
    idx = (-jnp.asarray(id_robot, jnp.int32)) % N
    wlin = jnp.reshape(w_lin, (1,)).astype(jnp.float32)

    xt = jnp.transpose(input, (2, 0, 1)).reshape(DIN, B * N)
    xr = jax.lax.dynamic_index_in_dim(input, idx, axis=1, keepdims=False)
    xrt = jnp.transpose(xr.reshape(B // BB, BB, DIN), (0, 2, 1))
    g = jnp.asarray(_G)

    wg4 = W_gat.reshape(M_EMB, G_H, F_OUT)
    ws_t = jnp.einsum("ehf,hf->he", wg4, a_src)
    wd_t = jnp.einsum("ehf,hf->he", wg4, a_dst)
    wsdT = jnp.concatenate([ws_t, wd_t], axis=0)

    grid_spec = pltpu.PrefetchScalarGridSpec(
        num_scalar_prefetch=1,
        grid=(B // BB,),
        in_specs=[
            pl.BlockSpec((DIN, BB * N), lambda i, *_: (0, i)),
            pl.BlockSpec((1, DIN, BB), lambda i, *_: (i, 0, 0)),
            pl.BlockSpec((BB, N), lambda i, *_: (i, 0)),
            pl.BlockSpec((ES_H, BB, N), lambda i, *_: (0, i, 0)),
            pl.BlockSpec((ES_EMB, DIN), lambda i, *_: (0, 0)),
            pl.BlockSpec((ES_H, ES_EMB), lambda i, *_: (0, 0)),
            pl.BlockSpec((M_EMB, ES_EMB), lambda i, *_: (0, 0)),
            pl.BlockSpec((G_H * F_OUT, M_EMB), lambda i, *_: (0, 0)),
            pl.BlockSpec((2 * G_H, M_EMB), lambda i, *_: (0, 0)),
            pl.BlockSpec((G_H * F_OUT, G_H), lambda i, *_: (0, 0)),
            pl.BlockSpec((BB * N, BB), lambda i, *_: (0, 0)),
        ],
        out_specs=pl.BlockSpec((1, G_H * F_OUT, BB), lambda i, *_: (i, 0, 0)),
    )
    out3 = pl.pallas_call(
        _lc_body,
        grid_spec=grid_spec,
        out_shape=jax.ShapeDtypeStruct((B // BB, G_H * F_OUT, BB), jnp.float32),
    )(wlin, xt, xrt, visibility, g, W_emb.T, Wq, W_out.T, W_gat.T, wsdT,
      jnp.asarray(_E), jnp.asarray(_R))
    return jnp.transpose(out3, (0, 2, 1)).reshape(B, G_H * F_OUT)
